# Initial kernel scaffold; baseline (speedup 1.0000x reference)
import functools

import jax
import jax.numpy as jnp
from jax import lax
from jax.experimental import pallas as pl
from jax.experimental.pallas import tpu as pltpu
from jax.experimental.pallas import tpu_sc as plsc

N = 100000
C = 64
K = 27
EK = 50000
TOT = K * EK
NW = 32
PER_W = 42240
PAD = NW * PER_W
CHUNK = 768
NBLK = CHUNK // 128
NCHUNK = PER_W // CHUNK
R = 25000
P = 4
SPR = 25088
ZCH = SPR // 16 // 2
DUMMY = 1 << 28

_mesh = plsc.VectorSubcoreMesh(core_axis_name="c", subcore_axis_name="s")


@functools.partial(
    pl.kernel,
    mesh=_mesh,
    out_type=jax.ShapeDtypeStruct((PAD, C), jnp.float32),
    scratch_types=[
        pltpu.VMEM((NBLK, 128), jnp.int32),
        pltpu.VMEM((CHUNK, C), jnp.float32),
        pltpu.SemaphoreType.DMA,
    ],
)
def _sc_gather(x_hbm, idx_hbm, out_hbm, idxbuf, rowbuf, sem):
    c = lax.axis_index("c")
    s = lax.axis_index("s")
    wid = c * 16 + s
    blkbase = wid * (PER_W // 128)

    def body(j, carry):
        blk = blkbase + j * NBLK
        off = blk * 128
        pltpu.sync_copy(idx_hbm.at[pl.ds(blk, NBLK), :], idxbuf)
        for b in range(NBLK):
            pltpu.async_copy(
                x_hbm.at[idxbuf.at[b]],
                rowbuf.at[pl.ds(b * 128, 128), :],
                sem,
            ).wait()
        pltpu.sync_copy(rowbuf, out_hbm.at[pl.ds(off, CHUNK), :])
        return carry

    lax.fori_loop(0, NCHUNK, body, 0)


@functools.partial(
    pl.kernel,
    mesh=_mesh,
    out_type=jax.ShapeDtypeStruct((2, N, C), jnp.float32),
    scratch_types=[
        pltpu.VMEM((NBLK, 128), jnp.int32),
        pltpu.VMEM((NBLK, 128), jnp.int32),
        pltpu.VMEM((CHUNK, C), jnp.float32),
        pltpu.VMEM((ZCH, C), jnp.float32),
        pltpu.VMEM_SHARED((SPR, C), jnp.float32),
    ],
)
def _sc_scatter(oidx_hbm, msg_hbm, zeros_hbm, part_hbm,
                oidxbuf, adjbuf, msgbuf, zerobuf, spmem):
    c = lax.axis_index("c")
    s = lax.axis_index("s")
    wid = c * 16 + s
    blkbase = wid * (PER_W // 128)
    pltpu.sync_copy(zeros_hbm, zerobuf)

    def phase(p, carry):
        plsc.subcore_barrier()
        pltpu.sync_copy(zerobuf, spmem.at[pl.ds(s * 2 * ZCH, ZCH), :])
        pltpu.sync_copy(zerobuf, spmem.at[pl.ds(s * 2 * ZCH + ZCH, ZCH), :])
        plsc.subcore_barrier()
        lo = p * R

        def chunk(j, carry2):
            blk = blkbase + j * NBLK
            off = blk * 128
            pltpu.sync_copy(oidx_hbm.at[pl.ds(blk, NBLK), :], oidxbuf)
            pltpu.sync_copy(msg_hbm.at[pl.ds(off, CHUNK), :], msgbuf)
            for b in range(NBLK):
                for i in range(128 // 16):
                    v = oidxbuf[b, pl.ds(i * 16, 16)]
                    loc = v - lo
                    ok = (loc >= 0) & (loc < R)
                    adjbuf[b, pl.ds(i * 16, 16)] = jnp.where(ok, loc, R)
                pltpu.sync_copy(
                    msgbuf.at[pl.ds(b * 128, 128), :],
                    spmem.at[adjbuf.at[b]],
                    add=True,
                )
            return carry2

        lax.fori_loop(0, NCHUNK, chunk, 0)
        plsc.subcore_barrier()

        @pl.when(s < 25)
        def _():
            pltpu.sync_copy(
                spmem.at[pl.ds(s * 1000, 1000), :],
                part_hbm.at[c, pl.ds(lo + s * 1000, 1000), :],
            )

        @pl.when(s + 16 < 25)
        def _():
            pltpu.sync_copy(
                spmem.at[pl.ds((s + 16) * 1000, 1000), :],
                part_hbm.at[c, pl.ds(lo + (s + 16) * 1000, 1000), :],
            )

        return carry

    lax.fori_loop(0, P, phase, 0)


def _mm_body(a_ref, w_ref, o_ref):
    o_ref[...] = lax.dot_general(
        a_ref[...], w_ref[0],
        (((1,), (0,)), ((), ())),
        precision=lax.Precision.HIGHEST,
        preferred_element_type=jnp.float32,
    )


_mm = pl.pallas_call(
    _mm_body,
    grid=(K, 25),
    in_specs=[
        pl.BlockSpec((1000, 128), lambda k, i: (k * 25 + i, 0)),
        pl.BlockSpec((1, 128, 128), lambda k, i: (k, 0, 0)),
    ],
    out_specs=pl.BlockSpec((1000, 128), lambda k, i: (k * 25 + i, 0)),
    out_shape=jax.ShapeDtypeStruct((PAD // 2, 128), jnp.float32),
)


def _add_body(a_ref, b_ref, o_ref):
    o_ref[...] = a_ref[...] + b_ref[...]


_addk = pl.pallas_call(
    _add_body,
    grid=(25,),
    in_specs=[
        pl.BlockSpec((4000, C), lambda i: (i, 0)),
        pl.BlockSpec((4000, C), lambda i: (i, 0)),
    ],
    out_specs=pl.BlockSpec((4000, C), lambda i: (i, 0)),
    out_shape=jax.ShapeDtypeStruct((N, C), jnp.float32),
)


def kernel(x, in_idx, out_idx, kernel):
    in_flat = in_idx.reshape(-1).astype(jnp.int32)
    out_flat = out_idx.reshape(-1).astype(jnp.int32)
    in_pad = jnp.concatenate(
        [in_flat, jnp.zeros((PAD - TOT,), jnp.int32)]).reshape(PAD // 128, 128)
    out_pad = jnp.concatenate(
        [out_flat, jnp.full((PAD - TOT,), DUMMY, jnp.int32)]
    ).reshape(PAD // 128, 128)
    w2 = jnp.zeros((K, 128, 128), jnp.float32)
    w2 = w2.at[:, :C, :C].set(kernel).at[:, C:, C:].set(kernel)
    zeros = jnp.zeros((ZCH, C), jnp.float32)

    gathered = _sc_gather(x, in_pad)
    msg2 = _mm(gathered.reshape(PAD // 2, 128), w2)
    parts = _sc_scatter(out_pad, msg2.reshape(PAD, C), zeros)
    return _addk(parts[0], parts[1])


# speedup vs baseline: 1.5986x; 1.5986x over previous
import functools

import jax
import jax.numpy as jnp
from jax import lax
from jax.experimental import pallas as pl
from jax.experimental.pallas import tpu as pltpu
from jax.experimental.pallas import tpu_sc as plsc

N = 100000
C = 64
K = 27
EK = 50000
TOT = K * EK
NW = 32
PER_W = 43008
PAD = NW * PER_W
GCH = 512
GNB = GCH // 128
GNC = PER_W // GCH
SCH = 256
SNB = SCH // 128
SNC = PER_W // SCH
RV = 20224
RP = RV // 2
P = 5
SPR = 10240
ZCH = 64
PPW = RP // 16
DUMMY = 1 << 28

_mesh = plsc.VectorSubcoreMesh(core_axis_name="c", subcore_axis_name="s")


@functools.partial(
    pl.kernel,
    mesh=_mesh,
    out_type=jax.ShapeDtypeStruct((PAD, 128), jnp.float32),
    scratch_types=[
        pltpu.VMEM((GCH,), jnp.int32),
        pltpu.VMEM((GCH, 128), jnp.float32),
        pltpu.SemaphoreType.DMA,
    ],
)
def _sc_gather(x2_hbm, idx_hbm, out_hbm, idxbuf, rowbuf, sem):
    c = lax.axis_index("c")
    s = lax.axis_index("s")
    wid = c * 16 + s
    base = wid * PER_W

    def body(j, carry):
        off = base + j * GCH
        pltpu.sync_copy(idx_hbm.at[pl.ds(off, GCH)], idxbuf)
        for b in range(GNB):
            pltpu.async_copy(
                x2_hbm.at[idxbuf.at[pl.ds(b * 128, 128)]],
                rowbuf.at[pl.ds(b * 128, 128), :],
                sem,
            ).wait()
        pltpu.sync_copy(rowbuf, out_hbm.at[pl.ds(off, GCH), :])
        return carry

    lax.fori_loop(0, GNC, body, 0)


@functools.partial(
    pl.kernel,
    mesh=_mesh,
    out_type=jax.ShapeDtypeStruct((2, P * RP, 128), jnp.float32),
    scratch_types=[
        pltpu.VMEM((SCH,), jnp.int32),
        pltpu.VMEM((SNB, 128), jnp.int32),
        pltpu.VMEM((SCH, 128), jnp.float32),
        pltpu.VMEM((ZCH, 128), jnp.float32),
        pltpu.VMEM_SHARED((SPR, 128), jnp.float32),
    ],
)
def _sc_scatter(oidx_hbm, msg_hbm, zeros_hbm, part_hbm,
                oidxbuf, adjbuf, msgbuf, zerobuf, spmem):
    c = lax.axis_index("c")
    s = lax.axis_index("s")
    wid = c * 16 + s
    base = wid * PER_W
    pltpu.sync_copy(zeros_hbm, zerobuf)

    def phase(p, carry):
        plsc.subcore_barrier()
        for z in range(10):
            pltpu.sync_copy(
                zerobuf, spmem.at[pl.ds((s * 10 + z) * ZCH, ZCH), :])
        plsc.subcore_barrier()
        lo = p * RV

        def chunk(j, carry2):
            off = base + j * SCH
            pltpu.sync_copy(oidx_hbm.at[pl.ds(off, SCH)], oidxbuf)
            pltpu.sync_copy(msg_hbm.at[pl.ds(off, SCH), :], msgbuf)
            for b in range(SNB):
                for i in range(128 // 16):
                    v = oidxbuf[pl.ds(b * 128 + i * 16, 16)]
                    loc = v - lo
                    ok = (loc >= 0) & (loc < RV)
                    pair = lax.shift_right_logical(loc, 1)
                    dummy = RP + (lax.shift_right_logical(v, 1) & 127)
                    adjbuf[b, pl.ds(i * 16, 16)] = jnp.where(ok, pair, dummy)
                pltpu.sync_copy(
                    msgbuf.at[pl.ds(b * 128, 128), :],
                    spmem.at[adjbuf.at[b]],
                    add=True,
                )
            return carry2

        lax.fori_loop(0, SNC, chunk, 0)
        plsc.subcore_barrier()

        pltpu.sync_copy(
            spmem.at[pl.ds(s * PPW, PPW), :],
            part_hbm.at[c, pl.ds(p * RP + s * PPW, PPW), :],
        )
        return carry

    lax.fori_loop(0, P, phase, 0)


def _mm_body(a_ref, w_ref, o_ref):
    o_ref[...] = lax.dot_general(
        a_ref[...], w_ref[0],
        (((1,), (0,)), ((), ())),
        precision=lax.Precision.HIGHEST,
        preferred_element_type=jnp.float32,
    )


_mm = pl.pallas_call(
    _mm_body,
    grid=(K, 25),
    in_specs=[
        pl.BlockSpec((2000, 128), lambda k, i: (k * 25 + i, 0)),
        pl.BlockSpec((1, 128, 128), lambda k, i: (k, 0, 0)),
    ],
    out_specs=pl.BlockSpec((2000, 128), lambda k, i: (k * 25 + i, 0)),
    out_shape=jax.ShapeDtypeStruct((PAD, 128), jnp.float32),
)


def _add_body(a_ref, b_ref, o_ref):
    o_ref[...] = a_ref[...] + b_ref[...]


_addk = pl.pallas_call(
    _add_body,
    grid=(40,),
    in_specs=[
        pl.BlockSpec((1264, 128), lambda i: (i, 0)),
        pl.BlockSpec((1264, 128), lambda i: (i, 0)),
    ],
    out_specs=pl.BlockSpec((1264, 128), lambda i: (i, 0)),
    out_shape=jax.ShapeDtypeStruct((P * RP, 128), jnp.float32),
)


def kernel(x, in_idx, out_idx, kernel):
    in_flat = in_idx.reshape(-1).astype(jnp.int32)
    out_flat = out_idx.reshape(-1).astype(jnp.int32)
    x2 = jnp.concatenate(
        [jnp.pad(x, ((0, 0), (0, 64))), jnp.pad(x, ((0, 0), (64, 0)))])
    gidx = in_flat + N * (out_flat & 1)
    ar = jnp.arange(PAD - TOT, dtype=jnp.int32)
    in_pad = jnp.concatenate([gidx, ar % N])
    out_pad = jnp.concatenate([out_flat, DUMMY + 2 * (ar % 128)])
    w2 = jnp.zeros((K, 128, 128), jnp.float32)
    w2 = w2.at[:, :C, :C].set(kernel).at[:, C:, C:].set(kernel)
    zeros = jnp.zeros((ZCH, 128), jnp.float32)

    gathered = _sc_gather(x2, in_pad)
    msg2 = _mm(gathered, w2)
    parts = _sc_scatter(out_pad, msg2, zeros)
    res = _addk(parts[0], parts[1])
    return res[:N // 2].reshape(N, C)


# speedup vs baseline: 1.7495x; 1.0944x over previous
import functools

import jax
import jax.numpy as jnp
from jax import lax
from jax.experimental import pallas as pl
from jax.experimental.pallas import tpu as pltpu
from jax.experimental.pallas import tpu_sc as plsc

N = 100000
C = 64
K = 27
EK = 50000
TOT = K * EK
NW = 32
PER_W = 43008
PAD = NW * PER_W
GCH = 384
GNB = GCH // 128
GNC = PER_W // GCH
SCH = 256
SNB = SCH // 128
SNC = PER_W // SCH
RV = 20224
RP = RV // 2
P = 5
SPR = 10240
ZCH = 64
PPW = RP // 16
DUMMY = 1 << 28

_mesh = plsc.VectorSubcoreMesh(core_axis_name="c", subcore_axis_name="s")


@functools.partial(
    pl.kernel,
    mesh=_mesh,
    out_type=jax.ShapeDtypeStruct((PAD, 128), jnp.float32),
    scratch_types=[
        pltpu.VMEM((GCH,), jnp.int32),
        pltpu.VMEM((GCH,), jnp.int32),
        pltpu.VMEM((GCH, 128), jnp.float32),
        pltpu.VMEM((GCH, 128), jnp.float32),
        pltpu.SemaphoreType.DMA,
        pltpu.SemaphoreType.DMA,
        pltpu.SemaphoreType.DMA,
        pltpu.SemaphoreType.DMA,
    ],
)
def _sc_gather(x2_hbm, idx_hbm, out_hbm,
               idxbuf0, idxbuf1, rowbuf0, rowbuf1, sg0, sg1, sw0, sw1):
    c = lax.axis_index("c")
    s = lax.axis_index("s")
    wid = c * 16 + s
    base = wid * PER_W

    def body(j, carry):
        off0 = base + (2 * j) * GCH
        off1 = off0 + GCH
        pltpu.sync_copy(idx_hbm.at[pl.ds(off0, GCH)], idxbuf0)

        @pl.when(j > 0)
        def _():
            pltpu.make_async_copy(
                rowbuf0, out_hbm.at[pl.ds(0, GCH), :], sw0).wait()

        g0 = [pltpu.async_copy(
            x2_hbm.at[idxbuf0.at[pl.ds(k * 128, 128)]],
            rowbuf0.at[pl.ds(k * 128, 128), :], sg0) for k in range(GNB)]
        pltpu.sync_copy(idx_hbm.at[pl.ds(off1, GCH)], idxbuf1)

        @pl.when(j > 0)
        def _():
            pltpu.make_async_copy(
                rowbuf1, out_hbm.at[pl.ds(0, GCH), :], sw1).wait()

        g1 = [pltpu.async_copy(
            x2_hbm.at[idxbuf1.at[pl.ds(k * 128, 128)]],
            rowbuf1.at[pl.ds(k * 128, 128), :], sg1) for k in range(GNB)]
        for h in g0:
            h.wait()
        pltpu.async_copy(rowbuf0, out_hbm.at[pl.ds(off0, GCH), :], sw0)
        for h in g1:
            h.wait()
        pltpu.async_copy(rowbuf1, out_hbm.at[pl.ds(off1, GCH), :], sw1)
        return carry

    lax.fori_loop(0, GNC // 2, body, 0)
    pltpu.make_async_copy(rowbuf0, out_hbm.at[pl.ds(0, GCH), :], sw0).wait()
    pltpu.make_async_copy(rowbuf1, out_hbm.at[pl.ds(0, GCH), :], sw1).wait()


@functools.partial(
    pl.kernel,
    mesh=_mesh,
    out_type=jax.ShapeDtypeStruct((2, P * RP, 128), jnp.float32),
    scratch_types=[
        pltpu.VMEM((SCH,), jnp.int32),
        pltpu.VMEM((SNB, 128), jnp.int32),
        pltpu.VMEM((SCH, 128), jnp.float32),
        pltpu.VMEM((ZCH, 128), jnp.float32),
        pltpu.VMEM_SHARED((SPR, 128), jnp.float32),
    ],
)
def _sc_scatter(oidx_hbm, msg_hbm, zeros_hbm, part_hbm,
                oidxbuf, adjbuf, msgbuf, zerobuf, spmem):
    c = lax.axis_index("c")
    s = lax.axis_index("s")
    wid = c * 16 + s
    base = wid * PER_W
    pltpu.sync_copy(zeros_hbm, zerobuf)

    def phase(p, carry):
        plsc.subcore_barrier()
        for z in range(10):
            pltpu.sync_copy(
                zerobuf, spmem.at[pl.ds((s * 10 + z) * ZCH, ZCH), :])
        plsc.subcore_barrier()
        lo = p * RV

        def chunk(j, carry2):
            off = base + j * SCH
            pltpu.sync_copy(oidx_hbm.at[pl.ds(off, SCH)], oidxbuf)
            pltpu.sync_copy(msg_hbm.at[pl.ds(off, SCH), :], msgbuf)
            for b in range(SNB):
                for i in range(128 // 16):
                    v = oidxbuf[pl.ds(b * 128 + i * 16, 16)]
                    loc = v - lo
                    ok = (loc >= 0) & (loc < RV)
                    pair = lax.shift_right_logical(loc, 1)
                    dummy = RP + (lax.shift_right_logical(v, 1) & 127)
                    adjbuf[b, pl.ds(i * 16, 16)] = jnp.where(ok, pair, dummy)
                pltpu.sync_copy(
                    msgbuf.at[pl.ds(b * 128, 128), :],
                    spmem.at[adjbuf.at[b]],
                    add=True,
                )
            return carry2

        lax.fori_loop(0, SNC, chunk, 0)
        plsc.subcore_barrier()

        pltpu.sync_copy(
            spmem.at[pl.ds(s * PPW, PPW), :],
            part_hbm.at[c, pl.ds(p * RP + s * PPW, PPW), :],
        )
        return carry

    lax.fori_loop(0, P, phase, 0)


def _mm_body(a_ref, w_ref, o_ref):
    o_ref[...] = lax.dot_general(
        a_ref[...].astype(jnp.bfloat16), w_ref[0],
        (((1,), (0,)), ((), ())),
        preferred_element_type=jnp.float32,
    )


_mm = pl.pallas_call(
    _mm_body,
    grid=(K, 25),
    in_specs=[
        pl.BlockSpec((2000, 128), lambda k, i: (k * 25 + i, 0)),
        pl.BlockSpec((1, 128, 128), lambda k, i: (k, 0, 0)),
    ],
    out_specs=pl.BlockSpec((2000, 128), lambda k, i: (k * 25 + i, 0)),
    out_shape=jax.ShapeDtypeStruct((PAD, 128), jnp.float32),

)


def _add_body(a_ref, b_ref, o_ref):
    o_ref[...] = a_ref[...] + b_ref[...]


_addk = pl.pallas_call(
    _add_body,
    grid=(40,),
    in_specs=[
        pl.BlockSpec((1264, 128), lambda i: (i, 0)),
        pl.BlockSpec((1264, 128), lambda i: (i, 0)),
    ],
    out_specs=pl.BlockSpec((1264, 128), lambda i: (i, 0)),
    out_shape=jax.ShapeDtypeStruct((P * RP, 128), jnp.float32),
)


def kernel(x, in_idx, out_idx, kernel):
    in_flat = in_idx.reshape(-1).astype(jnp.int32)
    out_flat = out_idx.reshape(-1).astype(jnp.int32)
    x2 = jnp.concatenate(
        [jnp.pad(x, ((0, 0), (0, 64))), jnp.pad(x, ((0, 0), (64, 0)))])
    gidx = in_flat + N * (out_flat & 1)
    ar = jnp.arange(PAD - TOT, dtype=jnp.int32)
    in_pad = jnp.concatenate([gidx, ar % N])
    out_pad = jnp.concatenate([out_flat, DUMMY + 2 * (ar % 128)])
    wb = kernel.astype(jnp.bfloat16)
    w2 = jnp.zeros((K, 128, 128), jnp.bfloat16)
    w2 = w2.at[:, :C, :C].set(wb).at[:, C:, C:].set(wb)
    zeros = jnp.zeros((ZCH, 128), jnp.float32)

    gathered = _sc_gather(x2, in_pad)
    msg2 = _mm(gathered, w2)
    parts = _sc_scatter(out_pad, msg2, zeros)
    res = _addk(parts[0], parts[1])
    return res[:N // 2].reshape(N, C)


# speedup vs baseline: 1.9598x; 1.1202x over previous
import functools

import jax
import jax.numpy as jnp
from jax import lax
from jax.experimental import pallas as pl
from jax.experimental.pallas import tpu as pltpu
from jax.experimental.pallas import tpu_sc as plsc

N = 100000
C = 64
K = 27
EK = 50000
TOT = K * EK
NW = 32
PER_W = 43008
PAD = NW * PER_W
GCH = 384
GNB = GCH // 128
GNC = PER_W // GCH
SEG = 2048
NSEG = PER_W // SEG
RV = 20224
RP = RV // 2
P = 5
SPR = 10240
ZCH = 64
PPW = RP // 16
DUMMY = 1 << 28

_mesh = plsc.VectorSubcoreMesh(core_axis_name="c", subcore_axis_name="s")


@functools.partial(
    pl.kernel,
    mesh=_mesh,
    out_type=jax.ShapeDtypeStruct((PAD, 128), jnp.float32),
    scratch_types=[
        pltpu.VMEM((GCH,), jnp.int32),
        pltpu.VMEM((GCH,), jnp.int32),
        pltpu.VMEM((GCH, 128), jnp.float32),
        pltpu.VMEM((GCH, 128), jnp.float32),
        pltpu.SemaphoreType.DMA,
        pltpu.SemaphoreType.DMA,
        pltpu.SemaphoreType.DMA,
        pltpu.SemaphoreType.DMA,
    ],
)
def _sc_gather(x2_hbm, idx_hbm, out_hbm,
               idxbuf0, idxbuf1, rowbuf0, rowbuf1, sg0, sg1, sw0, sw1):
    c = lax.axis_index("c")
    s = lax.axis_index("s")
    wid = c * 16 + s
    base = wid * PER_W

    def body(j, carry):
        off0 = base + (2 * j) * GCH
        off1 = off0 + GCH
        pltpu.sync_copy(idx_hbm.at[pl.ds(off0, GCH)], idxbuf0)

        @pl.when(j > 0)
        def _():
            pltpu.make_async_copy(
                rowbuf0, out_hbm.at[pl.ds(0, GCH), :], sw0).wait()

        g0 = [pltpu.async_copy(
            x2_hbm.at[idxbuf0.at[pl.ds(k * 128, 128)]],
            rowbuf0.at[pl.ds(k * 128, 128), :], sg0) for k in range(GNB)]
        pltpu.sync_copy(idx_hbm.at[pl.ds(off1, GCH)], idxbuf1)

        @pl.when(j > 0)
        def _():
            pltpu.make_async_copy(
                rowbuf1, out_hbm.at[pl.ds(0, GCH), :], sw1).wait()

        g1 = [pltpu.async_copy(
            x2_hbm.at[idxbuf1.at[pl.ds(k * 128, 128)]],
            rowbuf1.at[pl.ds(k * 128, 128), :], sg1) for k in range(GNB)]
        for h in g0:
            h.wait()
        pltpu.async_copy(rowbuf0, out_hbm.at[pl.ds(off0, GCH), :], sw0)
        for h in g1:
            h.wait()
        pltpu.async_copy(rowbuf1, out_hbm.at[pl.ds(off1, GCH), :], sw1)
        return carry

    lax.fori_loop(0, GNC // 2, body, 0)
    pltpu.make_async_copy(rowbuf0, out_hbm.at[pl.ds(0, GCH), :], sw0).wait()
    pltpu.make_async_copy(rowbuf1, out_hbm.at[pl.ds(0, GCH), :], sw1).wait()


@functools.partial(
    pl.kernel,
    mesh=_mesh,
    out_type=jax.ShapeDtypeStruct((2, P * RP, 128), jnp.float32),
    scratch_types=[
        pltpu.VMEM((256,), jnp.int32),
        pltpu.VMEM((1, 128), jnp.int32),
        pltpu.VMEM((1, 128), jnp.int32),
        pltpu.VMEM((128, 128), jnp.float32),
        pltpu.VMEM((128, 128), jnp.float32),
        pltpu.VMEM((ZCH, 128), jnp.float32),
        pltpu.VMEM_SHARED((SPR, 128), jnp.float32),
        pltpu.SemaphoreType.DMA,
        pltpu.SemaphoreType.DMA,
        pltpu.SemaphoreType.DMA,
        pltpu.SemaphoreType.DMA,
    ],
)
def _sc_scatter(oidx_hbm, msg_hbm, zeros_hbm, part_hbm,
                oidxbuf, clocA, clocB, stageA, stageB,
                zerobuf, spmem, sga, sgb, ssa, ssb):
    c = lax.axis_index("c")
    s = lax.axis_index("s")
    wid = c * 16 + s
    base = wid * PER_W
    pltpu.sync_copy(zeros_hbm, zerobuf)

    def phase(p, carry):
        plsc.subcore_barrier()
        for z in range(10):
            pltpu.sync_copy(
                zerobuf, spmem.at[pl.ds((s * 10 + z) * ZCH, ZCH), :])
        plsc.subcore_barrier()
        lo = p * RV

        def pair(j, carry2):
            off = base + j * 256
            pltpu.sync_copy(oidx_hbm.at[pl.ds(off, 256)], oidxbuf)

            @pl.when(j > 0)
            def _():
                pltpu.make_async_copy(
                    stageA, spmem.at[clocA.at[0]], ssa).wait()

            ga = pltpu.async_copy(
                msg_hbm.at[pl.ds(off, 128), :], stageA, sga)
            for i in range(8):
                v = oidxbuf[pl.ds(i * 16, 16)]
                loc = v - lo
                ok = (loc >= 0) & (loc < RV)
                pair_r = lax.shift_right_logical(loc, 1)
                dummy = RP + (lax.shift_right_logical(v, 1) & 127)
                clocA[0, pl.ds(i * 16, 16)] = jnp.where(ok, pair_r, dummy)

            @pl.when(j > 0)
            def _():
                pltpu.make_async_copy(
                    stageB, spmem.at[clocB.at[0]], ssb).wait()

            gb = pltpu.async_copy(
                msg_hbm.at[pl.ds(off + 128, 128), :], stageB, sgb)
            for i in range(8):
                v = oidxbuf[pl.ds(128 + i * 16, 16)]
                loc = v - lo
                ok = (loc >= 0) & (loc < RV)
                pair_r = lax.shift_right_logical(loc, 1)
                dummy = RP + (lax.shift_right_logical(v, 1) & 127)
                clocB[0, pl.ds(i * 16, 16)] = jnp.where(ok, pair_r, dummy)
            ga.wait()
            pltpu.async_copy(stageA, spmem.at[clocA.at[0]], ssa, add=True)
            gb.wait()
            pltpu.async_copy(stageB, spmem.at[clocB.at[0]], ssb, add=True)
            return carry2

        lax.fori_loop(0, PER_W // 256, pair, 0)
        pltpu.make_async_copy(stageA, spmem.at[clocA.at[0]], ssa).wait()
        pltpu.make_async_copy(stageB, spmem.at[clocB.at[0]], ssb).wait()
        plsc.subcore_barrier()

        pltpu.sync_copy(
            spmem.at[pl.ds(s * PPW, PPW), :],
            part_hbm.at[c, pl.ds(p * RP + s * PPW, PPW), :],
        )
        return carry

    lax.fori_loop(0, P, phase, 0)


def _mm_body(a_ref, w_ref, o_ref):
    o_ref[...] = lax.dot_general(
        a_ref[...].astype(jnp.bfloat16), w_ref[0],
        (((1,), (0,)), ((), ())),
        preferred_element_type=jnp.float32,
    )


_mm = pl.pallas_call(
    _mm_body,
    grid=(K, 25),
    in_specs=[
        pl.BlockSpec((2000, 128), lambda k, i: (k * 25 + i, 0)),
        pl.BlockSpec((1, 128, 128), lambda k, i: (k, 0, 0)),
    ],
    out_specs=pl.BlockSpec((2000, 128), lambda k, i: (k * 25 + i, 0)),
    out_shape=jax.ShapeDtypeStruct((PAD, 128), jnp.float32),

)


def _add_body(a_ref, b_ref, o_ref):
    o_ref[...] = a_ref[...] + b_ref[...]


_addk = pl.pallas_call(
    _add_body,
    grid=(40,),
    in_specs=[
        pl.BlockSpec((1264, 128), lambda i: (i, 0)),
        pl.BlockSpec((1264, 128), lambda i: (i, 0)),
    ],
    out_specs=pl.BlockSpec((1264, 128), lambda i: (i, 0)),
    out_shape=jax.ShapeDtypeStruct((P * RP, 128), jnp.float32),
)


def kernel(x, in_idx, out_idx, kernel):
    in_flat = in_idx.reshape(-1).astype(jnp.int32)
    out_flat = out_idx.reshape(-1).astype(jnp.int32)
    x2 = jnp.concatenate(
        [jnp.pad(x, ((0, 0), (0, 64))), jnp.pad(x, ((0, 0), (64, 0)))])
    gidx = in_flat + N * (out_flat & 1)
    ar = jnp.arange(PAD - TOT, dtype=jnp.int32)
    in_pad = jnp.concatenate([gidx, ar % N])
    out_pad = jnp.concatenate([out_flat, DUMMY + 2 * (ar % 128)])
    wb = kernel.astype(jnp.bfloat16)
    w2 = jnp.zeros((K, 128, 128), jnp.bfloat16)
    w2 = w2.at[:, :C, :C].set(wb).at[:, C:, C:].set(wb)
    zeros = jnp.zeros((ZCH, 128), jnp.float32)

    gathered = _sc_gather(x2, in_pad)
    msg2 = _mm(gathered, w2)
    parts = _sc_scatter(out_pad, msg2, zeros)
    res = _addk(parts[0], parts[1])
    return res[:N // 2].reshape(N, C)
